# baseline (speedup 1.0000x reference)
import functools

import jax
import jax.numpy as jnp
from jax import lax
from jax.experimental import pallas as pl
from jax.experimental.pallas import tpu as pltpu
from jax.experimental.pallas import tpu_sc as plsc

B = 64
C = 6625
T = 80
MAX_LEN = 25
LPAD = 32
TOTAL_Y = B * MAX_LEN

NTC = 40
NW = 32
LANES = 16
NFULL = C // LANES
CTAIL = NFULL * LANES
CPAD = 6640
TROWS = 8
NTCH = T // TROWS
TAB = 6656


def _sc_body(x_hbm, y_hbm, starts_hbm, lens_hbm, lut_hbm, out_hbm,
             xb_v, y_v, st_v, ln_v, lut_v, nk_v, yk_v, row_v, sem0, sem1):
    cid = lax.axis_index("c")
    sid = lax.axis_index("s")
    w = sid * 2 + cid
    b = NTC + w

    pltpu.sync_copy(y_hbm, y_v)
    pltpu.sync_copy(starts_hbm, st_v)
    pltpu.sync_copy(lens_hbm, ln_v)
    pltpu.sync_copy(lut_hbm, lut_v)

    zero16 = jnp.zeros((LANES,), jnp.float32)

    def _zero(i, carry):
        nk_v[pl.ds(i * LANES, LANES)] = zero16
        yk_v[pl.ds(i * LANES, LANES)] = zero16
        return carry

    lax.fori_loop(0, TAB // LANES, _zero, 0)

    lane_iota = lax.iota(jnp.int32, LANES)
    sems = (sem0, sem1)
    row_v[...] = zero16

    @pl.when(b < B)
    def _process():
        predvecs = [jnp.zeros((LANES,), jnp.int32) for _ in range(T // LANES)]

        pending = [None, None]
        pending[0] = pltpu.async_copy(
            x_hbm.at[b, pl.ds(0, TROWS), :], xb_v.at[0], sems[0])

        neg_inf = jnp.full((LANES,), -jnp.inf, jnp.float32)
        for ch in range(NTCH):
            slot = ch % 2
            if ch + 1 < NTCH:
                nxt = (ch + 1) % 2
                pending[nxt] = pltpu.async_copy(
                    x_hbm.at[b, pl.ds((ch + 1) * TROWS, TROWS), :],
                    xb_v.at[nxt], sems[nxt])
            pending[slot].wait()

            def class_body(i, carry, slot=slot):
                out = []
                ivec = jnp.full((LANES,), i, jnp.int32)
                for r in range(TROWS):
                    rm, ri = carry[2 * r], carry[2 * r + 1]
                    v = xb_v[slot, r, pl.ds(i * LANES, LANES)]
                    upd = v > rm
                    out.append(jnp.where(upd, v, rm))
                    out.append(jnp.where(upd, ivec, ri))
                return tuple(out)

            init = []
            for r in range(TROWS):
                init.append(neg_inf)
                init.append(jnp.zeros((LANES,), jnp.int32))
            carry = lax.fori_loop(0, NFULL, class_body, tuple(init),
                                  unroll=4)

            for r in range(TROWS):
                t = ch * TROWS + r
                rm, ri = carry[2 * r], carry[2 * r + 1]
                cls = ri * LANES + lane_iota
                vtail = xb_v[slot, r, pl.ds(C - LANES, LANES)]
                cls_tail = jnp.full((LANES,), C - LANES, jnp.int32) + lane_iota
                maxv = jnp.maximum(jnp.max(rm, axis=0),
                                   jnp.max(vtail, axis=0))
                maxvec = jnp.full((LANES,), maxv)
                cand = jnp.minimum(
                    jnp.where(rm == maxvec, cls, C),
                    jnp.where(vtail == maxvec, cls_tail, C))
                pred = jnp.min(cand, axis=0)
                predvecs[t // LANES] = jnp.where(
                    lane_iota == (t % LANES),
                    jnp.full((LANES,), pred), predvecs[t // LANES])

        one16 = jnp.full((LANES,), 1.0, jnp.float32)
        true16 = jnp.full((LANES,), True)
        for k in range(T // LANES):
            plsc.addupdate_scatter(nk_v, [predvecs[k]], one16, mask=true16)

        bvec = jnp.full((LANES,), b, jnp.int32)
        start = plsc.load_gather(st_v, [bvec])
        length = plsc.load_gather(ln_v, [bvec])

        labs, msks = [], []
        for g in range(2):
            off = lane_iota + g * LANES
            msk = off < length
            idx = jnp.minimum(start + off, TOTAL_Y - 1)
            lab = plsc.load_gather(y_v, [idx], mask=msk)
            lab = jnp.where(msk, lab, 0)
            labs.append(lab)
            msks.append(msk)
            plsc.addupdate_scatter(yk_v, [lab], one16, mask=msk)

        sum_nk = jnp.float32(0.0)
        ms_g, mult_g = [], []
        for g in range(2):
            mvals = plsc.load_gather(nk_v, [labs[g]], mask=msks[g])
            mults = plsc.load_gather(yk_v, [labs[g]], mask=msks[g])
            ms_g.append(mvals)
            mult_g.append(mults)
            sum_nk = sum_nk + jnp.sum(
                jnp.where(msks[g], mvals / mults, 0.0))

        log_l = plsc.load_gather(lut_v, [length])
        snk = jnp.full((LANES,), sum_nk, jnp.float32)
        loss = jnp.float32(0.0)
        for g in range(2):
            n_p = jnp.where(snk == 0.0, 1e-5,
                            jnp.maximum(ms_g[g] / snk, 1e-5))
            log_m = plsc.load_gather(
                lut_v, [mult_g[g].astype(jnp.int32)], mask=msks[g])
            contrib = jnp.where(
                msks[g], -n_p * (log_m - log_l) / mult_g[g], 0.0)
            loss = loss + jnp.sum(contrib)

        row_v[...] = jnp.where(lane_iota == 0,
                               jnp.full((LANES,), loss), zero16)

    pltpu.sync_copy(row_v, out_hbm.at[w])


def _tc_body(starts_ref, lens_ref, x_ref, y_ref, out_ref):
    b = pl.program_id(0)

    xb = x_ref[0]
    m = jnp.max(xb, axis=1, keepdims=True)
    lane_ids = jax.lax.broadcasted_iota(jnp.int32, (T, C), 1)
    cand = jnp.where(xb == m, lane_ids, C)
    predicts = jnp.min(cand, axis=1, keepdims=True)

    start = starts_ref[b]
    length = lens_ref[b]

    lane_row = jax.lax.broadcasted_iota(jnp.int32, (1, LPAD), 1)
    lab_row = jnp.full((1, LPAD), -1, jnp.int32)
    lab_scalars = []
    for j in range(LPAD):
        vj = y_ref[jnp.minimum(start + j, TOTAL_Y - 1)]
        labj = jnp.where(j < length, vj, -1)
        lab_scalars.append(labj)
        lab_row = jnp.where(lane_row == j, labj, lab_row)

    mult = jnp.zeros((1, LPAD), jnp.float32)
    for j in range(LPAD):
        mult += (lab_row == lab_scalars[j]).astype(jnp.float32)
    mcnt = jnp.sum((predicts == lab_row).astype(jnp.float32), axis=0,
                   keepdims=True)

    valid_row = lane_row < length
    validf = valid_row.astype(jnp.float32)
    inv_mult = validf / mult
    sum_nk = jnp.sum(mcnt * inv_mult, keepdims=True)[:, :1]

    n_p = jnp.where(sum_nk == 0.0, 1e-5, jnp.maximum(mcnt / sum_nk, 1e-5))
    log_yp = jnp.log(mult) - jnp.log(length.astype(jnp.float32))
    contrib = jnp.where(valid_row, -n_p * log_yp * inv_mult, 0.0)
    loss_b = jnp.sum(contrib, keepdims=True)[:, :1]

    @pl.when(b == 0)
    def _():
        out_ref[...] = jnp.zeros((1, 1), jnp.float32)

    out_ref[...] += loss_b


def _reduce_body(rows_ref, tc_ref, out_ref):
    out_ref[...] = (jnp.sum(rows_ref[...], keepdims=True)
                    + tc_ref[...]) * (1.0 / B)


@jax.jit
def kernel(x, y, target_lengths):
    ends = jnp.cumsum(target_lengths)
    starts = (ends - target_lengths).astype(jnp.int32)
    lens32 = target_lengths.astype(jnp.int32)
    lut = jnp.log(jnp.maximum(jnp.arange(32, dtype=jnp.float32), 1.0))
    xt = jnp.swapaxes(x, 1, 2)

    sc_call = functools.partial(
        pl.kernel,
        out_type=jax.ShapeDtypeStruct((NW, LANES), jnp.float32),
        mesh=plsc.VectorSubcoreMesh(core_axis_name="c", subcore_axis_name="s"),
        compiler_params=pltpu.CompilerParams(needs_layout_passes=False),
        scratch_types=[
            pltpu.VMEM((2, TROWS, C), jnp.float32),
            pltpu.VMEM((TOTAL_Y,), jnp.int32),
            pltpu.VMEM((B,), jnp.int32),
            pltpu.VMEM((B,), jnp.int32),
            pltpu.VMEM((32,), jnp.float32),
            pltpu.VMEM((TAB,), jnp.float32),
            pltpu.VMEM((TAB,), jnp.float32),
            pltpu.VMEM((LANES,), jnp.float32),
            pltpu.SemaphoreType.DMA,
            pltpu.SemaphoreType.DMA,
        ],
    )(_sc_body)
    rows = sc_call(xt, y, starts, lens32, lut)

    tc_part = pl.pallas_call(
        _tc_body,
        grid=(NTC,),
        in_specs=[
            pl.BlockSpec(memory_space=pltpu.SMEM),
            pl.BlockSpec(memory_space=pltpu.SMEM),
            pl.BlockSpec((1, T, C), lambda b: (b, 0, 0)),
            pl.BlockSpec(memory_space=pltpu.SMEM),
        ],
        out_specs=pl.BlockSpec((1, 1), lambda b: (0, 0)),
        out_shape=jax.ShapeDtypeStruct((1, 1), jnp.float32),
    )(starts, lens32, xt, y)

    out = pl.pallas_call(
        _reduce_body,
        out_shape=jax.ShapeDtypeStruct((1, 1), jnp.float32),
    )(rows, tc_part)
    return out[0, 0]


# speedup vs baseline: 1.0106x; 1.0106x over previous
import functools

import jax
import jax.numpy as jnp
from jax import lax
from jax.experimental import pallas as pl
from jax.experimental.pallas import tpu as pltpu
from jax.experimental.pallas import tpu_sc as plsc

B = 64
C = 6625
T = 80
MAX_LEN = 25
LPAD = 32
TOTAL_Y = B * MAX_LEN

NTC = 40
NW = 32
LANES = 16
NFULL = C // LANES
CTAIL = NFULL * LANES
CPAD = 6640
TROWS = 8
NTCH = T // TROWS
TAB = 6656


def _sc_body(x_hbm, y_hbm, starts_hbm, lens_hbm, lut_hbm, out_hbm,
             xb_v, y_v, st_v, ln_v, lut_v, nk_v, yk_v, row_v, sem0, sem1):
    cid = lax.axis_index("c")
    sid = lax.axis_index("s")
    w = sid * 2 + cid
    b = NTC + w

    pltpu.sync_copy(y_hbm, y_v)
    pltpu.sync_copy(starts_hbm, st_v)
    pltpu.sync_copy(lens_hbm, ln_v)
    pltpu.sync_copy(lut_hbm, lut_v)

    zero16 = jnp.zeros((LANES,), jnp.float32)

    def _zero(i, carry):
        nk_v[pl.ds(i * LANES, LANES)] = zero16
        yk_v[pl.ds(i * LANES, LANES)] = zero16
        return carry

    lax.fori_loop(0, TAB // LANES, _zero, 0)

    lane_iota = lax.iota(jnp.int32, LANES)
    sems = (sem0, sem1)
    row_v[...] = zero16

    @pl.when(b < B)
    def _process():
        predvecs = [jnp.zeros((LANES,), jnp.int32) for _ in range(T // LANES)]

        pending = [None, None]
        pending[0] = pltpu.async_copy(
            x_hbm.at[b, pl.ds(0, TROWS), :], xb_v.at[0], sems[0])

        neg_inf = jnp.full((LANES,), -jnp.inf, jnp.float32)
        for ch in range(NTCH):
            slot = ch % 2
            if ch + 1 < NTCH:
                nxt = (ch + 1) % 2
                pending[nxt] = pltpu.async_copy(
                    x_hbm.at[b, pl.ds((ch + 1) * TROWS, TROWS), :],
                    xb_v.at[nxt], sems[nxt])
            pending[slot].wait()

            def class_body(i, carry, slot=slot):
                out = []
                ivec = jnp.full((LANES,), i, jnp.int32)
                for r in range(TROWS):
                    rm, ri = carry[2 * r], carry[2 * r + 1]
                    v = xb_v[slot, r, pl.ds(i * LANES, LANES)]
                    upd = v > rm
                    out.append(jnp.where(upd, v, rm))
                    out.append(jnp.where(upd, ivec, ri))
                return tuple(out)

            init = []
            for r in range(TROWS):
                init.append(neg_inf)
                init.append(jnp.zeros((LANES,), jnp.int32))
            carry = lax.fori_loop(0, NFULL, class_body, tuple(init),
                                  unroll=4)

            for r in range(TROWS):
                t = ch * TROWS + r
                rm, ri = carry[2 * r], carry[2 * r + 1]
                cls = ri * LANES + lane_iota
                vtail = xb_v[slot, r, pl.ds(C - LANES, LANES)]
                cls_tail = jnp.full((LANES,), C - LANES, jnp.int32) + lane_iota
                maxv = jnp.maximum(jnp.max(rm, axis=0),
                                   jnp.max(vtail, axis=0))
                maxvec = jnp.full((LANES,), maxv)
                cand = jnp.minimum(
                    jnp.where(rm == maxvec, cls, C),
                    jnp.where(vtail == maxvec, cls_tail, C))
                pred = jnp.min(cand, axis=0)
                predvecs[t // LANES] = jnp.where(
                    lane_iota == (t % LANES),
                    jnp.full((LANES,), pred), predvecs[t // LANES])

        one16 = jnp.full((LANES,), 1.0, jnp.float32)
        true16 = jnp.full((LANES,), True)
        for k in range(T // LANES):
            plsc.addupdate_scatter(nk_v, [predvecs[k]], one16, mask=true16)

        bvec = jnp.full((LANES,), b, jnp.int32)
        start = plsc.load_gather(st_v, [bvec])
        length = plsc.load_gather(ln_v, [bvec])

        labs, msks = [], []
        for g in range(2):
            off = lane_iota + g * LANES
            msk = off < length
            lab = plsc.load_gather(y_v, [start + off], mask=msk)
            lab = jnp.where(msk, lab, 0)
            labs.append(lab)
            msks.append(msk)
            plsc.addupdate_scatter(yk_v, [lab], one16, mask=msk)

        sum_nk = jnp.float32(0.0)
        ms_g, mult_g = [], []
        for g in range(2):
            mvals = plsc.load_gather(nk_v, [labs[g]], mask=msks[g])
            mults = plsc.load_gather(yk_v, [labs[g]], mask=msks[g])
            ms_g.append(mvals)
            mult_g.append(mults)
            sum_nk = sum_nk + jnp.sum(
                jnp.where(msks[g], mvals / mults, 0.0))

        log_l = plsc.load_gather(lut_v, [length])
        snk = jnp.full((LANES,), sum_nk, jnp.float32)
        loss = jnp.float32(0.0)
        for g in range(2):
            n_p = jnp.where(snk == 0.0, 1e-5,
                            jnp.maximum(ms_g[g] / snk, 1e-5))
            log_m = plsc.load_gather(
                lut_v, [mult_g[g].astype(jnp.int32)], mask=msks[g])
            contrib = jnp.where(
                msks[g], -n_p * (log_m - log_l) / mult_g[g], 0.0)
            loss = loss + jnp.sum(contrib)

        row_v[...] = jnp.where(lane_iota == 0,
                               jnp.full((LANES,), loss), zero16)

    pltpu.sync_copy(row_v, out_hbm.at[w])


def _tc_body(starts_ref, lens_ref, x_ref, y_ref, out_ref):
    b = pl.program_id(0)

    xb = x_ref[0]
    m = jnp.max(xb, axis=1, keepdims=True)
    lane_ids = jax.lax.broadcasted_iota(jnp.int32, (T, C), 1)
    cand = jnp.where(xb == m, lane_ids, C)
    predicts = jnp.min(cand, axis=1, keepdims=True)

    start = starts_ref[b]
    length = lens_ref[b]

    lab = y_ref[pl.ds(start, LPAD), :]
    pos = jax.lax.broadcasted_iota(jnp.int32, (LPAD, 1), 0)
    valid_col = pos < length
    lab = jnp.where(valid_col, lab, -1)

    lab_b = jnp.broadcast_to(lab, (LPAD, LPAD))
    eye = (jax.lax.broadcasted_iota(jnp.int32, (LPAD, LPAD), 0)
           == jax.lax.broadcasted_iota(jnp.int32, (LPAD, LPAD), 1))
    lab_row = jnp.sum(jnp.where(eye, lab_b, 0), axis=0, keepdims=True)

    mult = jnp.sum((lab == lab_row).astype(jnp.float32), axis=0,
                   keepdims=True)
    mcnt = jnp.sum((predicts == lab_row).astype(jnp.float32), axis=0,
                   keepdims=True)

    valid_row = (jax.lax.broadcasted_iota(jnp.int32, (1, LPAD), 1)
                 < length)
    validf = valid_row.astype(jnp.float32)
    inv_mult = validf / mult
    sum_nk = jnp.sum(mcnt * inv_mult, keepdims=True)[:, :1]

    n_p = jnp.where(sum_nk == 0.0, 1e-5, jnp.maximum(mcnt / sum_nk, 1e-5))
    log_yp = jnp.log(mult) - jnp.log(length.astype(jnp.float32))
    contrib = jnp.where(valid_row, -n_p * log_yp * inv_mult, 0.0)
    loss_b = jnp.sum(contrib, keepdims=True)[:, :1]

    @pl.when(b == 0)
    def _():
        out_ref[...] = jnp.zeros((1, 1), jnp.float32)

    out_ref[...] += loss_b


def _reduce_body(rows_ref, tc_ref, out_ref):
    out_ref[...] = (jnp.sum(rows_ref[...], keepdims=True)
                    + tc_ref[...]) * (1.0 / B)


@jax.jit
def kernel(x, y, target_lengths):
    ends = jnp.cumsum(target_lengths)
    starts = (ends - target_lengths).astype(jnp.int32)
    lens32 = target_lengths.astype(jnp.int32)
    y_pad1 = jnp.zeros((TOTAL_Y + LPAD,), jnp.int32).at[:TOTAL_Y].set(y)
    lut = jnp.log(jnp.maximum(jnp.arange(32, dtype=jnp.float32), 1.0))
    xt = jnp.swapaxes(x, 1, 2)

    sc_call = functools.partial(
        pl.kernel,
        out_type=jax.ShapeDtypeStruct((NW, LANES), jnp.float32),
        mesh=plsc.VectorSubcoreMesh(core_axis_name="c", subcore_axis_name="s"),
        compiler_params=pltpu.CompilerParams(needs_layout_passes=False),
        scratch_types=[
            pltpu.VMEM((2, TROWS, C), jnp.float32),
            pltpu.VMEM((TOTAL_Y + LPAD,), jnp.int32),
            pltpu.VMEM((B,), jnp.int32),
            pltpu.VMEM((B,), jnp.int32),
            pltpu.VMEM((32,), jnp.float32),
            pltpu.VMEM((TAB,), jnp.float32),
            pltpu.VMEM((TAB,), jnp.float32),
            pltpu.VMEM((LANES,), jnp.float32),
            pltpu.SemaphoreType.DMA,
            pltpu.SemaphoreType.DMA,
        ],
    )(_sc_body)
    rows = sc_call(xt, y_pad1, starts, lens32, lut)

    y_pad2 = y_pad1.reshape(TOTAL_Y + LPAD, 1)
    tc_part = pl.pallas_call(
        _tc_body,
        grid=(NTC,),
        in_specs=[
            pl.BlockSpec(memory_space=pltpu.SMEM),
            pl.BlockSpec(memory_space=pltpu.SMEM),
            pl.BlockSpec((1, T, C), lambda b: (b, 0, 0)),
            pl.BlockSpec((TOTAL_Y + LPAD, 1), lambda b: (0, 0)),
        ],
        out_specs=pl.BlockSpec((1, 1), lambda b: (0, 0)),
        out_shape=jax.ShapeDtypeStruct((1, 1), jnp.float32),
    )(starts, lens32, xt, y_pad2)

    out = pl.pallas_call(
        _reduce_body,
        out_shape=jax.ShapeDtypeStruct((1, 1), jnp.float32),
    )(rows, tc_part)
    return out[0, 0]
